# baseline (speedup 1.0000x reference)
import jax
import jax.numpy as jnp
from jax.experimental import pallas as pl
from jax.experimental.pallas import tpu as pltpu


def _dice_body(smooth_ref, inp_ref, tgt_ref, out_ref, vacc_ref, r_ref):
    n = pl.program_id(0)
    hb = pl.program_id(1)
    num_n = pl.num_programs(0)
    num_hb = pl.num_programs(1)
    x = inp_ref[0]
    t = tgt_ref[0]
    smooth = smooth_ref[0, 0]
    C, BH, W = x.shape

    @pl.when((n == 0) & (hb == 0))
    def _init_r():
        r_ref[0] = 0.0

    for c in range(C):
        xc = x[c]
        eq = t == c
        pi = jnp.sum(jnp.where(eq, xc, 0.0), axis=0, keepdims=True)
        px = jnp.sum(xc, axis=0, keepdims=True)
        pc = jnp.sum(jnp.where(eq, 1.0, 0.0), axis=0, keepdims=True)
        row = jnp.concatenate([pi, px, pc], axis=0)

        @pl.when(hb == 0)
        def _set():
            vacc_ref[3 * c:3 * c + 3, :] = row

        @pl.when(hb != 0)
        def _add():
            vacc_ref[3 * c:3 * c + 3, :] += row

    @pl.when(hb == num_hb - 1)
    def _fold_sample():
        r = jnp.float32(0.0)
        for c in range(C):
            inter = jnp.sum(vacc_ref[3 * c])
            xsum = jnp.sum(vacc_ref[3 * c + 1])
            cnt = jnp.sum(vacc_ref[3 * c + 2])
            r = r + (2.0 * inter + smooth) / (xsum + cnt + smooth)
        r_ref[0] = r_ref[0] + r

        @pl.when(n == num_n - 1)
        def _fini():
            out_ref[0, 0] = 1.0 - r_ref[0] / (num_n * C)


def kernel(inputs, targets, smooth):
    N, C, H, W = inputs.shape
    HB = 2
    BH = H // HB
    t32 = targets.astype(jnp.int32)
    s = jnp.asarray(smooth, jnp.float32).reshape(1, 1)
    out = pl.pallas_call(
        _dice_body,
        grid=(N, HB),
        in_specs=[
            pl.BlockSpec(memory_space=pltpu.SMEM),
            pl.BlockSpec((1, C, BH, W), lambda n, h: (n, 0, h, 0)),
            pl.BlockSpec((1, BH, W), lambda n, h: (n, h, 0)),
        ],
        out_specs=pl.BlockSpec(memory_space=pltpu.SMEM),
        out_shape=jax.ShapeDtypeStruct((1, 1), jnp.float32),
        scratch_shapes=[
            pltpu.VMEM((3 * C, W), jnp.float32),
            pltpu.SMEM((1,), jnp.float32),
        ],
    )(s, inputs, t32)
    return out[0, 0]


# speedup vs baseline: 1.4174x; 1.4174x over previous
import jax
import jax.numpy as jnp
from jax.experimental import pallas as pl
from jax.experimental.pallas import tpu as pltpu

_STRIP = 8


def _dice_body(smooth_ref, inp_ref, tgt_ref, out_ref, r_ref):
    n = pl.program_id(0)
    num_n = pl.num_programs(0)
    smooth = smooth_ref[0, 0]
    C = inp_ref.shape[1]
    H = inp_ref.shape[2]
    W = inp_ref.shape[3]

    @pl.when(n == 0)
    def _init_r():
        r_ref[0] = 0.0

    one = jnp.float32(1.0)
    zero = jnp.float32(0.0)
    acc_i = [jnp.zeros((_STRIP, W), jnp.float32) for _ in range(C)]
    acc_d = [jnp.zeros((_STRIP, W), jnp.float32) for _ in range(C)]
    for s in range(0, H, _STRIP):
        tv = tgt_ref[0, pl.ds(s, _STRIP), :]
        for c in range(C):
            xv = inp_ref[0, c, pl.ds(s, _STRIP), :]
            eq = tv == c
            acc_i[c] = acc_i[c] + jnp.where(eq, xv, zero)
            acc_d[c] = acc_d[c] + (xv + jnp.where(eq, one, zero))

    r = jnp.float32(0.0)
    for c in range(C):
        inter = jnp.sum(acc_i[c])
        den = jnp.sum(acc_d[c])
        r = r + (2.0 * inter + smooth) / (den + smooth)
    r_ref[0] = r_ref[0] + r

    @pl.when(n == num_n - 1)
    def _fini():
        out_ref[0, 0] = 1.0 - r_ref[0] / (num_n * C)


def kernel(inputs, targets, smooth):
    N, C, H, W = inputs.shape
    t32 = targets.astype(jnp.int32)
    s = jnp.asarray(smooth, jnp.float32).reshape(1, 1)
    out = pl.pallas_call(
        _dice_body,
        grid=(N,),
        in_specs=[
            pl.BlockSpec(memory_space=pltpu.SMEM),
            pl.BlockSpec((1, C, H, W), lambda n: (n, 0, 0, 0)),
            pl.BlockSpec((1, H, W), lambda n: (n, 0, 0)),
        ],
        out_specs=pl.BlockSpec(memory_space=pltpu.SMEM),
        out_shape=jax.ShapeDtypeStruct((1, 1), jnp.float32),
        scratch_shapes=[pltpu.SMEM((1,), jnp.float32)],
    )(s, inputs, t32)
    return out[0, 0]
